# baseline (speedup 1.0000x reference)
import jax
import jax.numpy as jnp
from jax import lax
from jax.experimental import pallas as pl
from jax.experimental.pallas import tpu as pltpu, tpu_sc as plsc

N = 10000
D = 128
DH = D // 2
E = 320000
C = 125
NCH = E // C
NCORE = 2
NSUB = 16
CH_T = NCH // NSUB
EPT = E // NSUB
NPAD = 10240
NT = NPAD // NSUB
RCH = 80
NBUF = 4
NIDX = 2 * NBUF


def _sc_body(x_hbm, rowflat_hbm, row_hbm, col_hbm, z2d_hbm, z1d_hbm,
             out_hbm, y_hbm, hist_hbm,
             rowring, colring, rowflat, bufs, xbufs, hist, tmp_l,
             deg_l, dis_l,
             gsems, ssems, isems, xsems, wsems, dsem, acc_sp):
    c = lax.axis_index("c")
    s = lax.axis_index("s")

    pltpu.sync_copy(z1d_hbm, hist)
    pltpu.sync_copy(z2d_hbm, xbufs.at[0])
    for k in range(NT // RCH):
        pltpu.async_copy(xbufs.at[0], acc_sp.at[pl.ds(s * NT + k * RCH, RCH)],
                         dsem)
    pltpu.sync_copy(rowflat_hbm.at[pl.ds(s * EPT, EPT)], rowflat)
    for k in range(NT // RCH):
        pltpu.make_async_copy(
            xbufs.at[0], acc_sp.at[pl.ds(s * NT + k * RCH, RCH)], dsem).wait()
    plsc.subcore_barrier()

    ones16 = jnp.full((16,), 1.0, jnp.float32)

    def hloop(i, carry):
        idx16 = rowflat[pl.ds(i * 16, 16)]
        plsc.addupdate_scatter(hist, [idx16], ones16)
        return carry

    lax.fori_loop(0, EPT // 16, hloop, 0)
    pltpu.sync_copy(hist, hist_hbm.at[c].at[s])
    plsc.subcore_barrier()

    pltpu.async_copy(hist_hbm.at[c].at[0].at[pl.ds(s * NT, NT)],
                     tmp_l.at[0], xsems.at[0])
    for tt in range(NSUB):
        if tt + 1 < NSUB:
            pltpu.async_copy(hist_hbm.at[c].at[tt + 1].at[pl.ds(s * NT, NT)],
                             tmp_l.at[(tt + 1) % 2], xsems.at[(tt + 1) % 2])
        pltpu.make_async_copy(hist_hbm.at[c].at[tt].at[pl.ds(s * NT, NT)],
                              tmp_l.at[tt % 2], xsems.at[tt % 2]).wait()
        for i in range(NT // 16):
            prev = (deg_l[pl.ds(i * 16, 16)] if tt > 0
                    else jnp.zeros((16,), jnp.float32))
            deg_l[pl.ds(i * 16, 16)] = prev + tmp_l[tt % 2, pl.ds(i * 16, 16)]
    for i in range(NT // 16):
        v = deg_l[pl.ds(i * 16, 16)]
        bits = lax.bitcast_convert_type(v, jnp.int32)
        g = lax.bitcast_convert_type(jnp.int32(0x5F3759DF) - (bits >> 1),
                                     jnp.float32)
        for _ in range(3):
            g = g * (1.5 - 0.5 * v * g * g)
        dis_l[pl.ds(i * 16, 16)] = jnp.where(v > 0.5, g, 0.0)

    nych = jnp.where(s == NSUB - 1, (N - (NSUB - 1) * NT) // RCH, NT // RCH)

    def _xload(j, b):
        base = s * NT + j * RCH
        return pltpu.async_copy(
            x_hbm.at[pl.ds(base, RCH), pl.ds(c * DH, DH)], xbufs.at[b],
            xsems.at[b])

    def _xload_wait(j, b):
        base = s * NT + j * RCH
        pltpu.make_async_copy(
            x_hbm.at[pl.ds(base, RCH), pl.ds(c * DH, DH)], xbufs.at[b],
            xsems.at[b]).wait()

    def _scale(buf, koff):
        for rg in range(RCH // 16):
            v16 = dis_l[pl.ds(koff + rg * 16, 16)]
            for lr in range(16):
                r = rg * 16 + lr
                sc = v16[lr]
                for d in range(DH // 16):
                    buf[r, pl.ds(d * 16, 16)] = buf[r, pl.ds(d * 16, 16)] * sc

    @pl.when(nych > 0)
    def _():
        _xload(0, 0)

    @pl.when(nych > 1)
    def _():
        _xload(1, 1)

    def yloop(j, carry):
        b = lax.rem(j, 2)
        base = s * NT + j * RCH
        _xload_wait(j, b)
        _scale(xbufs.at[b], j * RCH)
        pltpu.async_copy(xbufs.at[b], y_hbm.at[c].at[pl.ds(base, RCH)],
                         wsems.at[b])
        pltpu.make_async_copy(xbufs.at[b], y_hbm.at[c].at[pl.ds(base, RCH)],
                              wsems.at[b]).wait()

        @pl.when(j + 2 < nych)
        def _():
            _xload(j + 2, b)
        return carry

    lax.fori_loop(0, nych, yloop, 0)
    plsc.subcore_barrier()

    def _iload(j, bi):
        pltpu.async_copy(row_hbm.at[s * CH_T + j], rowring.at[bi],
                         isems.at[bi])
        pltpu.async_copy(col_hbm.at[s * CH_T + j], colring.at[bi],
                         isems.at[bi])

    def _iload_wait(j, bi):
        pltpu.make_async_copy(row_hbm.at[s * CH_T + j], rowring.at[bi],
                              isems.at[bi]).wait()
        pltpu.make_async_copy(col_hbm.at[s * CH_T + j], colring.at[bi],
                              isems.at[bi]).wait()

    def _gather(b, bi):
        return pltpu.async_copy(y_hbm.at[c].at[colring.at[bi]], bufs.at[b],
                                gsems.at[b])

    def _gather_wait(b, bi):
        pltpu.make_async_copy(y_hbm.at[c].at[colring.at[bi]], bufs.at[b],
                              gsems.at[b]).wait()

    def _scat(b, bi):
        return pltpu.async_copy(bufs.at[b], acc_sp.at[rowring.at[bi]],
                                ssems.at[b], add=True)

    def _scat_wait(b, bi):
        pltpu.make_async_copy(bufs.at[b], acc_sp.at[rowring.at[bi]],
                              ssems.at[b]).wait()

    for b in range(NBUF):
        _iload(b, b)
    for b in range(NBUF):
        _iload_wait(b, b)
        _gather(b, b)
        _iload(NBUF + b, NBUF + b)

    def eloop(q, carry):
        for b in range(NBUF):
            j = q * NBUF + b
            bi = lax.rem(j, NIDX)
            _gather_wait(b, bi)
            _scat(b, bi)
        for b in range(NBUF):
            j = q * NBUF + b
            jn = j + NBUF
            bi = lax.rem(j, NIDX)
            bin_ = lax.rem(jn, NIDX)
            _scat_wait(b, bi)

            @pl.when(j + NIDX < CH_T)
            def _():
                _iload(j + NIDX, bi)

            @pl.when(jn < CH_T)
            def _():
                _iload_wait(jn, bin_)
                _gather(b, bin_)
        return carry

    lax.fori_loop(0, CH_T // NBUF, eloop, 0)
    plsc.subcore_barrier()

    ndr = jnp.where(s == NSUB - 1, (N - (NSUB - 1) * NT) // RCH, NT // RCH)

    def _aload(k, b):
        return pltpu.async_copy(acc_sp.at[pl.ds(s * NT + k * RCH, RCH)],
                                xbufs.at[b], xsems.at[b])

    def _aload_wait(k, b):
        pltpu.make_async_copy(acc_sp.at[pl.ds(s * NT + k * RCH, RCH)],
                              xbufs.at[b], xsems.at[b]).wait()

    _aload(0, 0)

    @pl.when(ndr > 1)
    def _():
        _aload(1, 1)

    def drain(k, carry):
        b = lax.rem(k, 2)
        base = s * NT + k * RCH
        _aload_wait(k, b)
        _scale(xbufs.at[b], k * RCH)
        pltpu.async_copy(
            xbufs.at[b], out_hbm.at[pl.ds(base, RCH), pl.ds(c * DH, DH)],
            wsems.at[b])
        pltpu.make_async_copy(
            xbufs.at[b], out_hbm.at[pl.ds(base, RCH), pl.ds(c * DH, DH)],
            wsems.at[b]).wait()

        @pl.when(k + 2 < ndr)
        def _():
            _aload(k + 2, b)
        return carry

    lax.fori_loop(0, ndr, drain, 0)


@jax.jit
def kernel(x, edge_index):
    row2d = edge_index[0].reshape(NCH, C)
    col2d = edge_index[1].reshape(NCH, C)
    z2d = jnp.zeros((RCH, DH), jnp.float32)
    z1d = jnp.zeros((NPAD,), jnp.float32)

    sc_fn = pl.kernel(
        _sc_body,
        out_type=(
            jax.ShapeDtypeStruct((N, D), jnp.float32),
            jax.ShapeDtypeStruct((NCORE, NPAD, DH), jnp.float32),
            jax.ShapeDtypeStruct((NCORE, NSUB, NPAD), jnp.float32),
        ),
        mesh=plsc.VectorSubcoreMesh(core_axis_name="c", subcore_axis_name="s"),
        compiler_params=pltpu.CompilerParams(use_tc_tiling_on_sc=False,
                                             needs_layout_passes=False),
        scratch_types=[
            pltpu.VMEM((NIDX, C), jnp.int32),
            pltpu.VMEM((NIDX, C), jnp.int32),
            pltpu.VMEM((EPT,), jnp.int32),
            pltpu.VMEM((NBUF, C, DH), jnp.float32),
            pltpu.VMEM((2, RCH, DH), jnp.float32),
            pltpu.VMEM((NPAD,), jnp.float32),
            pltpu.VMEM((2, NT), jnp.float32),
            pltpu.VMEM((NT,), jnp.float32),
            pltpu.VMEM((NT,), jnp.float32),
            pltpu.SemaphoreType.DMA((NBUF,)),
            pltpu.SemaphoreType.DMA((NBUF,)),
            pltpu.SemaphoreType.DMA((NIDX,)),
            pltpu.SemaphoreType.DMA((2,)),
            pltpu.SemaphoreType.DMA((2,)),
            pltpu.SemaphoreType.DMA,
            pltpu.VMEM_SHARED((NPAD, DH), jnp.float32),
        ],
    )
    out, _y, _h = sc_fn(x, edge_index[0], row2d, col2d, z2d, z1d)
    return out


# speedup vs baseline: 1.0764x; 1.0764x over previous
import jax
import jax.numpy as jnp
from jax import lax
from jax.experimental import pallas as pl
from jax.experimental.pallas import tpu as pltpu, tpu_sc as plsc

N = 10000
D = 128
DH = D // 2
E = 320000
C = 125
NCH = E // C
NCORE = 2
NSUB = 16
CH_T = NCH // NSUB
EPT = E // NSUB
NPAD = 10240
NT = NPAD // NSUB
RCH = 80
NBUF = 5
NIDX = 2 * NBUF


def _sc_body(x_hbm, rowflat_hbm, row_hbm, col_hbm, z2d_hbm, z1d_hbm,
             ones_hbm, out_hbm, y_hbm, hist_hbm,
             rowring, colring, rowflat, bufs, xbufs, tmp_l,
             deg_l, dis_l, onesv,
             gsems, ssems, isems, xsems, wsems, dsem, acc_sp, deg_sp):
    c = lax.axis_index("c")
    s = lax.axis_index("s")

    pltpu.sync_copy(z2d_hbm, xbufs.at[0])
    for k in range(NT // RCH):
        pltpu.async_copy(xbufs.at[0], acc_sp.at[pl.ds(s * NT + k * RCH, RCH)],
                         dsem)
    pltpu.sync_copy(rowflat_hbm.at[pl.ds(s * CH_T, CH_T)], rowflat)
    pltpu.sync_copy(z1d_hbm.at[pl.ds(s * NT, NT)], deg_sp.at[pl.ds(s * NT, NT)])
    pltpu.sync_copy(ones_hbm, onesv)
    for k in range(NT // RCH):
        pltpu.make_async_copy(
            xbufs.at[0], acc_sp.at[pl.ds(s * NT + k * RCH, RCH)], dsem).wait()
    plsc.subcore_barrier()

    DEGW = 8

    def dloop(j, carry):
        @pl.when(j < CH_T)
        def _():
            pltpu.async_copy(onesv, deg_sp.at[rowflat.at[j]], dsem, add=True)

        @pl.when(j >= DEGW)
        def _():
            jw = jnp.maximum(j - DEGW, 0)
            pltpu.make_async_copy(
                onesv, deg_sp.at[rowflat.at[jw]], dsem).wait()
        return carry

    lax.fori_loop(0, CH_T + DEGW, dloop, 0)
    plsc.subcore_barrier()

    pltpu.sync_copy(deg_sp.at[pl.ds(s * NT, NT)], deg_l)
    for i in range(NT // 16):
        v = deg_l[pl.ds(i * 16, 16)]
        bits = lax.bitcast_convert_type(v, jnp.int32)
        g = lax.bitcast_convert_type(jnp.int32(0x5F3759DF) - (bits >> 1),
                                     jnp.float32)
        for _ in range(3):
            g = g * (1.5 - 0.5 * v * g * g)
        dis_l[pl.ds(i * 16, 16)] = jnp.where(v > 0.5, g, 0.0)

    nych = jnp.where(s == NSUB - 1, (N - (NSUB - 1) * NT) // RCH, NT // RCH)

    def _xload(j, b):
        base = s * NT + j * RCH
        return pltpu.async_copy(
            x_hbm.at[pl.ds(base, RCH), pl.ds(c * DH, DH)], xbufs.at[b],
            xsems.at[b])

    def _xload_wait(j, b):
        base = s * NT + j * RCH
        pltpu.make_async_copy(
            x_hbm.at[pl.ds(base, RCH), pl.ds(c * DH, DH)], xbufs.at[b],
            xsems.at[b]).wait()

    def _scale(buf, koff):
        for rg in range(RCH // 16):
            v16 = dis_l[pl.ds(koff + rg * 16, 16)]
            for lr in range(16):
                r = rg * 16 + lr
                sc = v16[lr]
                for d in range(DH // 16):
                    buf[r, pl.ds(d * 16, 16)] = buf[r, pl.ds(d * 16, 16)] * sc

    @pl.when(nych > 0)
    def _():
        _xload(0, 0)

    @pl.when(nych > 1)
    def _():
        _xload(1, 1)

    def yloop(j, carry):
        b = lax.rem(j, 2)
        base = s * NT + j * RCH
        _xload_wait(j, b)
        _scale(xbufs.at[b], j * RCH)
        pltpu.async_copy(xbufs.at[b], y_hbm.at[c].at[pl.ds(base, RCH)],
                         wsems.at[b])
        pltpu.make_async_copy(xbufs.at[b], y_hbm.at[c].at[pl.ds(base, RCH)],
                              wsems.at[b]).wait()

        @pl.when(j + 2 < nych)
        def _():
            _xload(j + 2, b)
        return carry

    lax.fori_loop(0, nych, yloop, 0)
    plsc.subcore_barrier()

    def _iload(j, bi):
        pltpu.async_copy(row_hbm.at[s * CH_T + j], rowring.at[bi],
                         isems.at[bi])
        pltpu.async_copy(col_hbm.at[s * CH_T + j], colring.at[bi],
                         isems.at[bi])

    def _iload_wait(j, bi):
        pltpu.make_async_copy(row_hbm.at[s * CH_T + j], rowring.at[bi],
                              isems.at[bi]).wait()
        pltpu.make_async_copy(col_hbm.at[s * CH_T + j], colring.at[bi],
                              isems.at[bi]).wait()

    def _gather(b, bi):
        return pltpu.async_copy(y_hbm.at[c].at[colring.at[bi]], bufs.at[b],
                                gsems.at[b])

    def _gather_wait(b, bi):
        pltpu.make_async_copy(y_hbm.at[c].at[colring.at[bi]], bufs.at[b],
                              gsems.at[b]).wait()

    def _scat(b, bi):
        return pltpu.async_copy(bufs.at[b], acc_sp.at[rowring.at[bi]],
                                ssems.at[b], add=True)

    def _scat_wait(b, bi):
        pltpu.make_async_copy(bufs.at[b], acc_sp.at[rowring.at[bi]],
                              ssems.at[b]).wait()

    for b in range(NBUF):
        _iload(b, b)
    for b in range(NBUF):
        _iload_wait(b, b)
        _gather(b, b)
        _iload(NBUF + b, NBUF + b)

    def eloop(q, carry):
        for b in range(NBUF):
            j = q * NBUF + b
            bi = lax.rem(j, NIDX)
            _gather_wait(b, bi)
            _scat(b, bi)
        for b in range(NBUF):
            j = q * NBUF + b
            jn = j + NBUF
            bi = lax.rem(j, NIDX)
            bin_ = lax.rem(jn, NIDX)
            _scat_wait(b, bi)

            @pl.when(j + NIDX < CH_T)
            def _():
                _iload(j + NIDX, bi)

            @pl.when(jn < CH_T)
            def _():
                _iload_wait(jn, bin_)
                _gather(b, bin_)
        return carry

    lax.fori_loop(0, CH_T // NBUF, eloop, 0)
    plsc.subcore_barrier()

    ndr = jnp.where(s == NSUB - 1, (N - (NSUB - 1) * NT) // RCH, NT // RCH)

    def _aload(k, b):
        return pltpu.async_copy(acc_sp.at[pl.ds(s * NT + k * RCH, RCH)],
                                xbufs.at[b], xsems.at[b])

    def _aload_wait(k, b):
        pltpu.make_async_copy(acc_sp.at[pl.ds(s * NT + k * RCH, RCH)],
                              xbufs.at[b], xsems.at[b]).wait()

    _aload(0, 0)

    @pl.when(ndr > 1)
    def _():
        _aload(1, 1)

    def drain(k, carry):
        b = lax.rem(k, 2)
        base = s * NT + k * RCH
        _aload_wait(k, b)
        _scale(xbufs.at[b], k * RCH)
        pltpu.async_copy(
            xbufs.at[b], out_hbm.at[pl.ds(base, RCH), pl.ds(c * DH, DH)],
            wsems.at[b])
        pltpu.make_async_copy(
            xbufs.at[b], out_hbm.at[pl.ds(base, RCH), pl.ds(c * DH, DH)],
            wsems.at[b]).wait()

        @pl.when(k + 2 < ndr)
        def _():
            _aload(k + 2, b)
        return carry

    lax.fori_loop(0, ndr, drain, 0)


@jax.jit
def kernel(x, edge_index):
    row2d = edge_index[0].reshape(NCH, C)
    col2d = edge_index[1].reshape(NCH, C)
    z2d = jnp.zeros((RCH, DH), jnp.float32)
    z1d = jnp.zeros((NPAD,), jnp.float32)
    ones = jnp.ones((C,), jnp.float32)

    sc_fn = pl.kernel(
        _sc_body,
        out_type=(
            jax.ShapeDtypeStruct((N, D), jnp.float32),
            jax.ShapeDtypeStruct((NCORE, NPAD, DH), jnp.float32),
            jax.ShapeDtypeStruct((NCORE, NSUB, NPAD), jnp.float32),
        ),
        mesh=plsc.VectorSubcoreMesh(core_axis_name="c", subcore_axis_name="s"),
        compiler_params=pltpu.CompilerParams(use_tc_tiling_on_sc=False,
                                             needs_layout_passes=False),
        scratch_types=[
            pltpu.VMEM((NIDX, C), jnp.int32),
            pltpu.VMEM((NIDX, C), jnp.int32),
            pltpu.VMEM((CH_T, C), jnp.int32),
            pltpu.VMEM((NBUF, C, DH), jnp.float32),
            pltpu.VMEM((2, RCH, DH), jnp.float32),
            pltpu.VMEM((2, NT), jnp.float32),
            pltpu.VMEM((NT,), jnp.float32),
            pltpu.VMEM((NT,), jnp.float32),
            pltpu.VMEM((C,), jnp.float32),
            pltpu.SemaphoreType.DMA((NBUF,)),
            pltpu.SemaphoreType.DMA((NBUF,)),
            pltpu.SemaphoreType.DMA((NIDX,)),
            pltpu.SemaphoreType.DMA((2,)),
            pltpu.SemaphoreType.DMA((2,)),
            pltpu.SemaphoreType.DMA,
            pltpu.VMEM_SHARED((NPAD, DH), jnp.float32),
            pltpu.VMEM_SHARED((NPAD,), jnp.float32),
        ],
    )
    out, _y, _h = sc_fn(x, row2d, row2d, col2d, z2d, z1d, ones)
    return out


# speedup vs baseline: 1.0778x; 1.0013x over previous
import jax
import jax.numpy as jnp
from jax import lax
from jax.experimental import pallas as pl
from jax.experimental.pallas import tpu as pltpu, tpu_sc as plsc

N = 10000
D = 128
DH = D // 2
E = 320000
C = 125
NCH = E // C
NCORE = 2
NSUB = 16
CH_T = NCH // NSUB
EPT = E // NSUB
NPAD = 10240
NT = NPAD // NSUB
RCH = 80
NBUF = 5
NIDX = 2 * NBUF


def _sc_body(x_hbm, rowflat_hbm, row_hbm, col_hbm, z2d_hbm, z1d_hbm,
             ones_hbm, out_hbm, y_hbm, hist_hbm,
             rowring, colring, rowflat, bufs, xbufs, tmp_l,
             deg_l, dis_l, onesv,
             gsems, ssems, isems, xsems, wsems, dsem, acc_sp, deg_sp):
    c = lax.axis_index("c")
    s = lax.axis_index("s")

    pltpu.sync_copy(z2d_hbm, xbufs.at[0])
    for k in range(NT // RCH):
        pltpu.async_copy(xbufs.at[0], acc_sp.at[pl.ds(s * NT + k * RCH, RCH)],
                         dsem)
    pltpu.sync_copy(rowflat_hbm.at[pl.ds(s * CH_T, CH_T)], rowflat)
    pltpu.sync_copy(z1d_hbm.at[pl.ds(0, NT)], tmp_l.at[0])
    pltpu.sync_copy(tmp_l.at[0], deg_sp.at[pl.ds(s * NT, NT)])
    pltpu.sync_copy(ones_hbm, onesv)
    for k in range(NT // RCH):
        pltpu.make_async_copy(
            xbufs.at[0], acc_sp.at[pl.ds(s * NT + k * RCH, RCH)], dsem).wait()
    plsc.subcore_barrier()

    DEGW = 8

    def dloop(j, carry):
        @pl.when(j < CH_T)
        def _():
            pltpu.async_copy(onesv, deg_sp.at[rowflat.at[j]], dsem, add=True)

        @pl.when(j >= DEGW)
        def _():
            jw = jnp.maximum(j - DEGW, 0)
            pltpu.make_async_copy(
                onesv, deg_sp.at[rowflat.at[jw]], dsem).wait()
        return carry

    lax.fori_loop(0, CH_T + DEGW, dloop, 0)
    plsc.subcore_barrier()

    pltpu.sync_copy(deg_sp.at[pl.ds(s * NT, NT)], deg_l)
    for i in range(NT // 16):
        v = deg_l[pl.ds(i * 16, 16)]
        bits = lax.bitcast_convert_type(v, jnp.int32)
        g = lax.bitcast_convert_type(jnp.int32(0x5F3759DF) - (bits >> 1),
                                     jnp.float32)
        for _ in range(3):
            g = g * (1.5 - 0.5 * v * g * g)
        dis_l[pl.ds(i * 16, 16)] = jnp.where(v > 0.5, g, 0.0)

    nych = jnp.where(s == NSUB - 1, (N - (NSUB - 1) * NT) // RCH, NT // RCH)

    def _xload(j, b):
        base = s * NT + j * RCH
        return pltpu.async_copy(
            x_hbm.at[pl.ds(base, RCH), pl.ds(c * DH, DH)], xbufs.at[b],
            xsems.at[b])

    def _xload_wait(j, b):
        base = s * NT + j * RCH
        pltpu.make_async_copy(
            x_hbm.at[pl.ds(base, RCH), pl.ds(c * DH, DH)], xbufs.at[b],
            xsems.at[b]).wait()

    def _scale(buf, koff):
        for rg in range(RCH // 16):
            v16 = dis_l[pl.ds(koff + rg * 16, 16)]
            for lr in range(16):
                r = rg * 16 + lr
                sc = v16[lr]
                for d in range(DH // 16):
                    buf[r, pl.ds(d * 16, 16)] = buf[r, pl.ds(d * 16, 16)] * sc

    @pl.when(nych > 0)
    def _():
        _xload(0, 0)

    @pl.when(nych > 1)
    def _():
        _xload(1, 1)

    def yloop(j, carry):
        b = lax.rem(j, 2)
        base = s * NT + j * RCH
        _xload_wait(j, b)
        _scale(xbufs.at[b], j * RCH)
        pltpu.async_copy(xbufs.at[b], y_hbm.at[c].at[pl.ds(base, RCH)],
                         wsems.at[b])
        pltpu.make_async_copy(xbufs.at[b], y_hbm.at[c].at[pl.ds(base, RCH)],
                              wsems.at[b]).wait()

        @pl.when(j + 2 < nych)
        def _():
            _xload(j + 2, b)
        return carry

    lax.fori_loop(0, nych, yloop, 0)
    plsc.subcore_barrier()

    def _iload(j, bi):
        pltpu.async_copy(row_hbm.at[s * CH_T + j], rowring.at[bi],
                         isems.at[bi])
        pltpu.async_copy(col_hbm.at[s * CH_T + j], colring.at[bi],
                         isems.at[bi])

    def _iload_wait(j, bi):
        pltpu.make_async_copy(row_hbm.at[s * CH_T + j], rowring.at[bi],
                              isems.at[bi]).wait()
        pltpu.make_async_copy(col_hbm.at[s * CH_T + j], colring.at[bi],
                              isems.at[bi]).wait()

    def _gather(b, bi):
        return pltpu.async_copy(y_hbm.at[c].at[colring.at[bi]], bufs.at[b],
                                gsems.at[b])

    def _gather_wait(b, bi):
        pltpu.make_async_copy(y_hbm.at[c].at[colring.at[bi]], bufs.at[b],
                              gsems.at[b]).wait()

    def _scat(b, bi):
        return pltpu.async_copy(bufs.at[b], acc_sp.at[rowring.at[bi]],
                                ssems.at[b], add=True)

    def _scat_wait(b, bi):
        pltpu.make_async_copy(bufs.at[b], acc_sp.at[rowring.at[bi]],
                              ssems.at[b]).wait()

    for b in range(NBUF):
        _iload(b, b)
    for b in range(NBUF):
        _iload_wait(b, b)
        _gather(b, b)
        _iload(NBUF + b, NBUF + b)

    def eloop(q, carry):
        for b in range(NBUF):
            j = q * NBUF + b
            bi = lax.rem(j, NIDX)
            _gather_wait(b, bi)
            _scat(b, bi)
        for b in range(NBUF):
            j = q * NBUF + b
            jn = j + NBUF
            bi = lax.rem(j, NIDX)
            bin_ = lax.rem(jn, NIDX)
            _scat_wait(b, bi)

            @pl.when(j + NIDX < CH_T)
            def _():
                _iload(j + NIDX, bi)

            @pl.when(jn < CH_T)
            def _():
                _iload_wait(jn, bin_)
                _gather(b, bin_)
        return carry

    lax.fori_loop(0, CH_T // NBUF, eloop, 0)
    plsc.subcore_barrier()

    ndr = jnp.where(s == NSUB - 1, (N - (NSUB - 1) * NT) // RCH, NT // RCH)

    def _aload(k, b):
        return pltpu.async_copy(acc_sp.at[pl.ds(s * NT + k * RCH, RCH)],
                                xbufs.at[b], xsems.at[b])

    def _aload_wait(k, b):
        pltpu.make_async_copy(acc_sp.at[pl.ds(s * NT + k * RCH, RCH)],
                              xbufs.at[b], xsems.at[b]).wait()

    _aload(0, 0)

    @pl.when(ndr > 1)
    def _():
        _aload(1, 1)

    def drain(k, carry):
        b = lax.rem(k, 2)
        base = s * NT + k * RCH
        _aload_wait(k, b)
        _scale(xbufs.at[b], k * RCH)
        pltpu.async_copy(
            xbufs.at[b], out_hbm.at[pl.ds(base, RCH), pl.ds(c * DH, DH)],
            wsems.at[b])
        pltpu.make_async_copy(
            xbufs.at[b], out_hbm.at[pl.ds(base, RCH), pl.ds(c * DH, DH)],
            wsems.at[b]).wait()

        @pl.when(k + 2 < ndr)
        def _():
            _aload(k + 2, b)
        return carry

    lax.fori_loop(0, ndr, drain, 0)


@jax.jit
def kernel(x, edge_index):
    row2d = edge_index[0].reshape(NCH, C)
    col2d = edge_index[1].reshape(NCH, C)
    z2d = jnp.zeros((RCH, DH), jnp.float32)
    z1d = jnp.zeros((NPAD,), jnp.float32)
    ones = jnp.ones((C,), jnp.float32)

    sc_fn = pl.kernel(
        _sc_body,
        out_type=(
            jax.ShapeDtypeStruct((N, D), jnp.float32),
            jax.ShapeDtypeStruct((NCORE, NPAD, DH), jnp.float32),
            jax.ShapeDtypeStruct((NCORE, NSUB, NPAD), jnp.float32),
        ),
        mesh=plsc.VectorSubcoreMesh(core_axis_name="c", subcore_axis_name="s"),
        compiler_params=pltpu.CompilerParams(use_tc_tiling_on_sc=False,
                                             needs_layout_passes=False),
        scratch_types=[
            pltpu.VMEM((NIDX, C), jnp.int32),
            pltpu.VMEM((NIDX, C), jnp.int32),
            pltpu.VMEM((CH_T, C), jnp.int32),
            pltpu.VMEM((NBUF, C, DH), jnp.float32),
            pltpu.VMEM((2, RCH, DH), jnp.float32),
            pltpu.VMEM((2, NT), jnp.float32),
            pltpu.VMEM((NT,), jnp.float32),
            pltpu.VMEM((NT,), jnp.float32),
            pltpu.VMEM((C,), jnp.float32),
            pltpu.SemaphoreType.DMA((NBUF,)),
            pltpu.SemaphoreType.DMA((NBUF,)),
            pltpu.SemaphoreType.DMA((NIDX,)),
            pltpu.SemaphoreType.DMA((2,)),
            pltpu.SemaphoreType.DMA((2,)),
            pltpu.SemaphoreType.DMA,
            pltpu.VMEM_SHARED((NPAD, DH), jnp.float32),
            pltpu.VMEM_SHARED((NPAD,), jnp.float32),
        ],
    )
    out, _y, _h = sc_fn(x, row2d, row2d, col2d, z2d, z1d, ones)
    return out
